# Initial kernel scaffold; baseline (speedup 1.0000x reference)
import jax
import jax.numpy as jnp
from jax.experimental import pallas as pl


def kernel(x, memory):
    raise NotImplementedError("write your pallas kernel here")


# speedup vs baseline: 1.2021x; 1.2021x over previous
import functools

import jax
import jax.numpy as jnp
from jax.experimental import pallas as pl

MEMC = 1024
CHUNK = 512


def _memnorm_body(mem_ref, out_ref):
    m = mem_ref[...]
    n = jnp.sqrt(jnp.sum(m * m, axis=1, keepdims=True))
    out_ref[...] = m / jnp.maximum(n, 1e-12)


def _vq_body(x_ref, mn_ref, mem_ref, o_ref):
    xc = x_ref[0]
    n = jnp.sqrt(jnp.sum(xc * xc, axis=0, keepdims=True))
    xn = xc / jnp.maximum(n, 1e-12)
    sim = jax.lax.dot_general(xn, mn_ref[...], (((0,), (1,)), ((), ())))
    idx = jnp.argmax(sim, axis=1)
    k_iota = jax.lax.broadcasted_iota(jnp.int32, (CHUNK, MEMC), 1)
    oh = (k_iota == idx[:, None]).astype(jnp.float32)
    o_ref[0] = jax.lax.dot_general(mem_ref[...], oh, (((0,), (1,)), ((), ())))


@jax.jit
def kernel(x, memory):
    B, C, H, W = x.shape
    HW = H * W
    x3 = x.reshape(B, C, HW)

    mem_norm = pl.pallas_call(
        _memnorm_body,
        out_shape=jax.ShapeDtypeStruct(memory.shape, memory.dtype),
    )(memory)

    grid = (B, HW // CHUNK)
    out = pl.pallas_call(
        _vq_body,
        grid=grid,
        in_specs=[
            pl.BlockSpec((1, C, CHUNK), lambda b, t: (b, 0, t)),
            pl.BlockSpec((MEMC, C), lambda b, t: (0, 0)),
            pl.BlockSpec((MEMC, C), lambda b, t: (0, 0)),
        ],
        out_specs=pl.BlockSpec((1, C, CHUNK), lambda b, t: (b, 0, t)),
        out_shape=jax.ShapeDtypeStruct((B, C, HW), x.dtype),
    )(x3, mem_norm, memory)

    return out.reshape(B, C, H, W)


# speedup vs baseline: 1.4230x; 1.1837x over previous
import functools

import jax
import jax.numpy as jnp
from jax.experimental import pallas as pl

MEMC = 1024
CHUNK = 1024


def _vq_body(x_ref, mn_ref, mem_ref, o_ref):
    xc = x_ref[0]
    n = jnp.sqrt(jnp.sum(xc * xc, axis=0, keepdims=True))
    xn = xc / jnp.maximum(n, 1e-12)
    sim = jax.lax.dot_general(xn, mn_ref[...], (((0,), (1,)), ((), ())))
    idx = jnp.argmax(sim, axis=1)
    k_iota = jax.lax.broadcasted_iota(jnp.int32, (CHUNK, MEMC), 1)
    oh = (k_iota == idx[:, None]).astype(jnp.float32)
    o_ref[0] = jax.lax.dot_general(mem_ref[...], oh, (((0,), (1,)), ((), ())))


@jax.jit
def kernel(x, memory):
    B, C, H, W = x.shape
    HW = H * W
    x3 = x.reshape(B, C, HW)

    mn = jnp.linalg.norm(memory, axis=1, keepdims=True)
    mem_norm = memory / jnp.maximum(mn, 1e-12)

    grid = (B, HW // CHUNK)
    out = pl.pallas_call(
        _vq_body,
        grid=grid,
        in_specs=[
            pl.BlockSpec((1, C, CHUNK), lambda b, t: (b, 0, t)),
            pl.BlockSpec((MEMC, C), lambda b, t: (0, 0)),
            pl.BlockSpec((MEMC, C), lambda b, t: (0, 0)),
        ],
        out_specs=pl.BlockSpec((1, C, CHUNK), lambda b, t: (b, 0, t)),
        out_shape=jax.ShapeDtypeStruct((B, C, HW), x.dtype),
    )(x3, mem_norm, memory)

    return out.reshape(B, C, H, W)


# speedup vs baseline: 1.5937x; 1.1200x over previous
import functools

import jax
import jax.numpy as jnp
from jax.experimental import pallas as pl

MEMC = 1024
CHUNK = 2048


def _vq_body(x_ref, mn_ref, mem_ref, o_ref):
    xc = x_ref[0]
    n = jnp.sqrt(jnp.sum(xc * xc, axis=0, keepdims=True))
    xn = xc / jnp.maximum(n, 1e-12)
    sim = jax.lax.dot_general(xn, mn_ref[...], (((0,), (1,)), ((), ())))
    idx = jnp.argmax(sim, axis=1)
    k_iota = jax.lax.broadcasted_iota(jnp.int32, (CHUNK, MEMC), 1)
    oh = (k_iota == idx[:, None]).astype(jnp.float32)
    o_ref[0] = jax.lax.dot_general(mem_ref[...], oh, (((0,), (1,)), ((), ())))


@jax.jit
def kernel(x, memory):
    B, C, H, W = x.shape
    HW = H * W
    x3 = x.reshape(B, C, HW)

    mn = jnp.linalg.norm(memory, axis=1, keepdims=True)
    mem_norm = memory / jnp.maximum(mn, 1e-12)

    grid = (B, HW // CHUNK)
    out = pl.pallas_call(
        _vq_body,
        grid=grid,
        in_specs=[
            pl.BlockSpec((1, C, CHUNK), lambda b, t: (b, 0, t)),
            pl.BlockSpec((MEMC, C), lambda b, t: (0, 0)),
            pl.BlockSpec((MEMC, C), lambda b, t: (0, 0)),
        ],
        out_specs=pl.BlockSpec((1, C, CHUNK), lambda b, t: (b, 0, t)),
        out_shape=jax.ShapeDtypeStruct((B, C, HW), x.dtype),
    )(x3, mem_norm, memory)

    return out.reshape(B, C, H, W)


# speedup vs baseline: 1.7431x; 1.0938x over previous
import functools

import jax
import jax.numpy as jnp
from jax.experimental import pallas as pl

MEMC = 1024
CHUNK = 4096


def _vq_body(x_ref, mn_ref, mem_ref, o_ref):
    xc = x_ref[0]
    n = jnp.sqrt(jnp.sum(xc * xc, axis=0, keepdims=True))
    xn = xc / jnp.maximum(n, 1e-12)
    sim = jax.lax.dot_general(xn, mn_ref[...], (((0,), (1,)), ((), ())))
    idx = jnp.argmax(sim, axis=1)
    k_iota = jax.lax.broadcasted_iota(jnp.int32, (CHUNK, MEMC), 1)
    oh = (k_iota == idx[:, None]).astype(jnp.float32)
    o_ref[0] = jax.lax.dot_general(mem_ref[...], oh, (((0,), (1,)), ((), ())))


@jax.jit
def kernel(x, memory):
    B, C, H, W = x.shape
    HW = H * W
    x3 = x.reshape(B, C, HW)

    mn = jnp.linalg.norm(memory, axis=1, keepdims=True)
    mem_norm = memory / jnp.maximum(mn, 1e-12)

    grid = (B, HW // CHUNK)
    out = pl.pallas_call(
        _vq_body,
        grid=grid,
        in_specs=[
            pl.BlockSpec((1, C, CHUNK), lambda b, t: (b, 0, t)),
            pl.BlockSpec((MEMC, C), lambda b, t: (0, 0)),
            pl.BlockSpec((MEMC, C), lambda b, t: (0, 0)),
        ],
        out_specs=pl.BlockSpec((1, C, CHUNK), lambda b, t: (b, 0, t)),
        out_shape=jax.ShapeDtypeStruct((B, C, HW), x.dtype),
    )(x3, mem_norm, memory)

    return out.reshape(B, C, H, W)
